# Initial kernel scaffold; baseline (speedup 1.0000x reference)
import jax
import jax.numpy as jnp
from jax.experimental import pallas as pl


def kernel(x, W, b):
    raise NotImplementedError("write your pallas kernel here")


# speedup vs baseline: 1.9195x; 1.9195x over previous
import jax
import jax.numpy as jnp
from jax.experimental import pallas as pl
from jax.experimental.pallas import tpu as pltpu

B = 128
D = 2048


def _gwg_kernel(x_ref, W_ref, b_ref, g_ref, u_ref, out_ref):
    x = x_ref[:]
    W = W_ref[:]
    b = b_ref[:]
    g = g_ref[:]
    u = u_ref[:]

    G = jnp.dot(x, W, preferred_element_type=jnp.float32) + b
    s = 1.0 - 2.0 * x
    fd = 0.5 * s * G

    t = fd + g
    tmax = jnp.max(t, axis=1, keepdims=True)
    col = jax.lax.broadcasted_iota(jnp.int32, (B, D), 1)
    idx = jnp.min(jnp.where(t == tmax, col, D), axis=1, keepdims=True)
    changes = (col == idx).astype(jnp.float32)

    mf = jnp.max(fd, axis=1, keepdims=True)
    lse_f = mf[:, 0] + jnp.log(jnp.sum(jnp.exp(fd - mf), axis=1))
    fd_i = jnp.sum(changes * fd, axis=1)
    lp_fwd = fd_i - lse_f

    w_row = jnp.dot(changes, W, preferred_element_type=jnp.float32)
    w_ii = jnp.sum(changes * w_row, axis=1)
    s_i = jnp.sum(changes * s, axis=1)
    G_i = jnp.sum(changes * G, axis=1)

    Gp = G + s_i[:, None] * w_row
    sp = s * (1.0 - 2.0 * changes)
    rd = 0.5 * sp * Gp
    mr = jnp.max(rd, axis=1, keepdims=True)
    lse_r = mr[:, 0] + jnp.log(jnp.sum(jnp.exp(rd - mr), axis=1))
    rd_i = jnp.sum(changes * rd, axis=1)
    lp_rev = rd_i - lse_r

    m_term = s_i * G_i + 0.5 * w_ii
    la = m_term + lp_rev - lp_fwd
    a = (jnp.exp(la) > u[:, 0]).astype(jnp.float32)
    out_ref[:] = x + (a[:, None] * changes) * s


def kernel(x, W, b):
    key = jax.random.key(42)
    ks, ku = jax.random.split(key)
    g = jax.random.gumbel(ks, x.shape, x.dtype)
    u = jax.random.uniform(ku, (x.shape[0],), x.dtype)
    return pl.pallas_call(
        _gwg_kernel,
        out_shape=jax.ShapeDtypeStruct((B, D), jnp.float32),
    )(x, W, b.reshape(1, D), g, u.reshape(B, 1))
